# Initial kernel scaffold; baseline (speedup 1.0000x reference)
import jax
import jax.numpy as jnp
from jax.experimental import pallas as pl


def kernel(x, edge_index, W1, b1, W2, b2, Wl, bl):
    raise NotImplementedError("write your pallas kernel here")


# speedup vs baseline: 36.9394x; 36.9394x over previous
import dataclasses
import functools

import jax
import jax.numpy as jnp
from jax import lax
from jax.experimental import pallas as pl
from jax.experimental.pallas import tpu as pltpu
from jax.experimental.pallas import tpu_sc as plsc

NC = 2
NS = 16
NW = NC * NS
CH = 128
LANES = 16

_vector_mesh = plsc.VectorSubcoreMesh(
    core_axis_name="core", subcore_axis_name="subcore")

_sc_params = pltpu.CompilerParams(
    needs_layout_passes=False, use_tc_tiling_on_sc=False)


def _hist_sc(dst1d, npad, tile_e):
    @functools.partial(
        pl.kernel,
        out_type=jax.ShapeDtypeStruct((NW, npad), jnp.float32),
        mesh=_vector_mesh,
        compiler_params=_sc_params,
        scratch_types=[
            pltpu.VMEM((tile_e,), jnp.int32),
            pltpu.VMEM((npad,), jnp.float32),
        ],
    )
    def hist_k(dst_hbm, out_hbm, idx_v, hist_v):
        c = lax.axis_index("core")
        s = lax.axis_index("subcore")
        w = c * NS + s
        pltpu.sync_copy(dst_hbm.at[pl.ds(w * tile_e, tile_e)], idx_v)

        @pl.loop(0, npad, step=LANES)
        def _(i):
            hist_v[pl.ds(i, LANES)] = jnp.zeros((LANES,), jnp.float32)

        ones = jnp.ones((LANES,), jnp.float32)

        @pl.loop(0, tile_e, step=LANES)
        def _(e):
            idx = idx_v[pl.ds(e, LANES)]
            plsc.addupdate_scatter(hist_v, [idx], ones)

        pltpu.sync_copy(hist_v, out_hbm.at[w])

    return hist_k(dst1d)


def _prop_sc(g, src2d, dst2d, npad, kr, rpt):
    hd = g.shape[1]

    @functools.partial(
        pl.kernel,
        out_type=jax.ShapeDtypeStruct((NC, npad, hd), jnp.float32),
        mesh=_vector_mesh,
        compiler_params=_sc_params,
        scratch_types=[
            pltpu.VMEM((kr, CH), jnp.int32),
            pltpu.VMEM((kr, CH), jnp.int32),
            pltpu.VMEM((CH, hd), jnp.float32),
            pltpu.VMEM((CH, hd), jnp.float32),
            pltpu.VMEM((rpt, hd), jnp.float32),
            pltpu.VMEM_SHARED((npad, hd), jnp.float32),
            pltpu.SemaphoreType.DMA,
            pltpu.SemaphoreType.DMA,
        ],
    )
    def prop_k(g_hbm, src_hbm, dst_hbm, out_hbm,
               srcv, dstv, buf0, buf1, stage, accum, sem0, sem1):
        c = lax.axis_index("core")
        s = lax.axis_index("subcore")
        w = c * NS + s

        cp_s = pltpu.async_copy(src_hbm.at[pl.ds(w * kr, kr)], srcv, sem0)
        cp_d = pltpu.async_copy(dst_hbm.at[pl.ds(w * kr, kr)], dstv, sem1)

        @pl.loop(0, rpt)
        def _(i):
            stage[i, :] = jnp.zeros((hd,), jnp.float32)

        pltpu.sync_copy(stage, accum.at[pl.ds(s * rpt, rpt)])
        cp_s.wait()
        cp_d.wait()
        plsc.subcore_barrier()

        pltpu.async_copy(g_hbm.at[srcv.at[0]], buf0, sem0)

        @pl.loop(0, kr, step=2)
        def _(j):
            pltpu.async_copy(g_hbm.at[srcv.at[j + 1]], buf1, sem1)
            pltpu.make_async_copy(g_hbm.at[srcv.at[0]], buf0, sem0).wait()
            pltpu.sync_copy(buf0, accum.at[dstv.at[j]], add=True)

            @pl.when(j + 2 < kr)
            def _():
                pltpu.async_copy(g_hbm.at[srcv.at[j + 2]], buf0, sem0)

            pltpu.make_async_copy(g_hbm.at[srcv.at[0]], buf1, sem1).wait()
            pltpu.sync_copy(buf1, accum.at[dstv.at[j + 1]], add=True)

        plsc.subcore_barrier()
        pltpu.sync_copy(accum.at[pl.ds(s * rpt, rpt)], stage)
        pltpu.sync_copy(stage, out_hbm.at[c, pl.ds(s * rpt, rpt)])

    return prop_k(g, src2d, dst2d)


def _tc_dis(hist, hd):
    nw, npad = hist.shape

    def body(h_ref, dis_ref):
        deg = jnp.sum(h_ref[...], axis=0) + 1.0
        dis = lax.rsqrt(deg)
        dis_ref[...] = jnp.broadcast_to(dis[:, None], (npad, hd))

    return pl.pallas_call(
        body,
        in_specs=[pl.BlockSpec((nw, npad), lambda: (0, 0))],
        out_specs=pl.BlockSpec((npad, hd), lambda: (0, 0)),
        out_shape=jax.ShapeDtypeStruct((npad, hd), jnp.float32),
    )(hist)


def _tc1(x, w1, dis, bn):
    n, f = x.shape
    hd = w1.shape[1]
    grid = n // bn

    def body(x_ref, w_ref, dis_ref, g_ref):
        h = jnp.dot(x_ref[...], w_ref[...], preferred_element_type=jnp.float32)
        g_ref[...] = h * dis_ref[...]

    return pl.pallas_call(
        body,
        grid=(grid,),
        in_specs=[
            pl.BlockSpec((bn, f), lambda i: (i, 0)),
            pl.BlockSpec((f, hd), lambda i: (0, 0)),
            pl.BlockSpec((bn, hd), lambda i: (i, 0)),
        ],
        out_specs=pl.BlockSpec((bn, hd), lambda i: (i, 0)),
        out_shape=jax.ShapeDtypeStruct((n, hd), jnp.float32),
    )(x, w1, dis)


def _tc2(p, g, dis, w2, b1, bn):
    n, hd = g.shape
    grid = n // bn

    def body(p_ref, g_ref, dis_ref, w_ref, b_ref, o_ref):
        s = p_ref[0] + p_ref[1] + g_ref[...]
        a = jnp.maximum(dis_ref[...] * s + b_ref[...], 0.0)
        h = jnp.dot(a, w_ref[...], preferred_element_type=jnp.float32)
        o_ref[...] = h * dis_ref[...]

    return pl.pallas_call(
        body,
        grid=(grid,),
        in_specs=[
            pl.BlockSpec((NC, bn, hd), lambda i: (0, i, 0)),
            pl.BlockSpec((bn, hd), lambda i: (i, 0)),
            pl.BlockSpec((bn, hd), lambda i: (i, 0)),
            pl.BlockSpec((hd, hd), lambda i: (0, 0)),
            pl.BlockSpec((1, hd), lambda i: (0, 0)),
        ],
        out_specs=pl.BlockSpec((bn, hd), lambda i: (i, 0)),
        out_shape=jax.ShapeDtypeStruct((n, hd), jnp.float32),
    )(p, g, dis, w2, b1)


def _tc3(q, g2, dis, b2, wl, bl, bn):
    n, hd = g2.shape
    co = wl.shape[1]
    grid = n // bn

    def body(q_ref, g_ref, dis_ref, b_ref, w_ref, bl_ref, o_ref):
        s = q_ref[0] + q_ref[1] + g_ref[...]
        a = jnp.maximum(dis_ref[...] * s + b_ref[...], 0.0)
        logits = jnp.dot(a, w_ref[...], preferred_element_type=jnp.float32)
        logits = logits + bl_ref[...]
        m = jnp.max(logits, axis=1, keepdims=True)
        lse = m + jnp.log(jnp.sum(jnp.exp(logits - m), axis=1, keepdims=True))
        o_ref[...] = logits - lse

    return pl.pallas_call(
        body,
        grid=(grid,),
        in_specs=[
            pl.BlockSpec((NC, bn, hd), lambda i: (0, i, 0)),
            pl.BlockSpec((bn, hd), lambda i: (i, 0)),
            pl.BlockSpec((bn, hd), lambda i: (i, 0)),
            pl.BlockSpec((1, hd), lambda i: (0, 0)),
            pl.BlockSpec((hd, co), lambda i: (0, 0)),
            pl.BlockSpec((1, co), lambda i: (0, 0)),
        ],
        out_specs=pl.BlockSpec((bn, co), lambda i: (i, 0)),
        out_shape=jax.ShapeDtypeStruct((n, co), jnp.float32),
    )(q, g2, dis, b2, wl, bl)


def kernel(x, edge_index, W1, b1, W2, b2, Wl, bl):
    n = x.shape[0]
    e = edge_index.shape[1]

    tile_e = -(-e // NW)
    tile_e = -(-tile_e // (2 * CH)) * (2 * CH)
    ep = tile_e * NW
    kr = tile_e // CH
    npad = -(-(n + 1) // (8 * NS)) * (8 * NS)
    rpt = npad // NS

    src = edge_index[0]
    dst = edge_index[1]
    pad = ep - e
    srcp = jnp.concatenate([src, jnp.zeros((pad,), jnp.int32)])
    dstp = jnp.concatenate([dst, jnp.full((pad,), n, jnp.int32)])
    src2d = srcp.reshape(NW * kr, CH)
    dst2d = dstp.reshape(NW * kr, CH)

    hist = _hist_sc(dstp, npad, tile_e)

    bn = 1000 if n % 1000 == 0 else 8
    hd = W1.shape[1]
    dis = _tc_dis(hist, hd)[:n]
    g1 = _tc1(x, W1, dis, bn)
    p = _prop_sc(g1, src2d, dst2d, npad, kr, rpt)
    g2 = _tc2(p, g1, dis, W2, b1.reshape(1, -1), bn)
    q = _prop_sc(g2, src2d, dst2d, npad, kr, rpt)
    return _tc3(q, g2, dis, b2.reshape(1, -1), Wl, bl.reshape(1, -1), bn)
